# baseline (speedup 1.0000x reference)
import functools

import numpy as np
import jax
import jax.numpy as jnp
from jax import lax
from jax.experimental import pallas as pl
from jax.experimental.pallas import tpu as pltpu
from jax.experimental.pallas import tpu_sc as plsc

_H = 112
_B = 32
_CT = 48
_EPS = 1e-08
_SEG = 28
_ROWS_PER_B = _CT * _H * 4
_NCHUNK = 16
_CHUNK = _ROWS_PER_B // _NCHUNK


def _build_cell_masks():
    g = np.zeros((16, _H, _H), dtype=np.float32)
    for i in range(16):
        hb, wb = divmod(i, 4)
        g[i, 28 * hb:28 * (hb + 1), 28 * wb:28 * (wb + 1)] = 1.0
    return g


def _build_cellmap448():
    m = np.empty((448,), dtype=np.int32)
    for lp in range(448):
        h, wband = divmod(lp, 4)
        m[lp] = (h // 28) * 4 + wband
    return m


_CELLS = _build_cell_masks()
_CELLMAP = _build_cellmap448()


def _imdiff_body(x_ref, o_ref):
    x = x_ref[0]
    vals = []
    for t in range(15):
        d = (jnp.abs(x[t] - x[t + 1]) + jnp.abs(x[16 + t] - x[17 + t])) \
            + jnp.abs(x[32 + t] - x[33 + t])
        vals.append(d)
    while len(vals) > 1:
        nxt = [vals[i] + vals[i + 1] for i in range(0, len(vals) - 1, 2)]
        if len(vals) % 2:
            nxt.append(vals[-1])
        vals = nxt
    o_ref[0] = vals[0] * np.float32(1.0 / 15.0)


def _gaussian_kernel1d(ksize, sigma):
    x = jnp.arange(ksize, dtype=jnp.float32) - (ksize // 2)
    g = jnp.exp(-(x ** 2) / (2.0 * sigma * sigma))
    return g / g.sum()


def _gauss_blur(img, ksize, sigma):
    k1 = _gaussian_kernel1d(ksize, sigma)
    k2 = jnp.outer(k1, k1)
    k2 = k2 / k2.sum()
    pad = ksize // 2
    x = jnp.pad(img, ((0, 0), (0, 0), (pad, pad), (pad, pad)), mode='reflect')
    kern = k2[None, None, :, :]
    return jax.lax.conv_general_dilated(x, kern, (1, 1), 'VALID',
                                        dimension_numbers=('NCHW', 'OIHW', 'NCHW'))


def _ni_batch(m):
    b, h, w = m.shape
    f = m.reshape(b, -1)
    f = f - f.min(axis=-1, keepdims=True)
    f = f / (f.max(axis=-1, keepdims=True) + _EPS)
    return f.reshape(b, h, w)


def _sc_route(x6, src_hbm, out, src_v, buf0, buf1, gs0, gs1, ws0, ws1):
    nc = 2
    b = lax.axis_index("s") * nc + lax.axis_index("c")
    pltpu.sync_copy(src_hbm.at[b], src_v)
    srcs = src_v[pl.ds(0, 16)]

    def do_cell(cell, buf, gsem, wsem):
        hb, wb = divmod(cell, 4)
        src = srcs[cell]
        g = pltpu.make_async_copy(x6.at[src, :, hb, :, wb, :], buf, gsem)
        g.start()
        g.wait()
        w = pltpu.make_async_copy(buf, out.at[b, :, hb, :, wb, :], wsem)
        w.start()
        return w

    for pair in range(8):
        w0 = do_cell(2 * pair, buf0, gs0, ws0)
        w1 = do_cell(2 * pair + 1, buf1, gs1, ws1)
        w0.wait()
        w1.wait()


def kernel(video_clips):
    b, c, t, h, w = video_clips.shape
    x = video_clips.reshape(b, c * t, h, w)
    grid_cells = jnp.asarray(_CELLS)

    im_diff = pl.pallas_call(
        _imdiff_body,
        grid=(b,),
        in_specs=[pl.BlockSpec((1, _CT, _H, _H), lambda i: (i, 0, 0, 0))],
        out_specs=pl.BlockSpec((1, _H, _H), lambda i: (i, 0, 0)),
        out_shape=jax.ShapeDtypeStruct((b, _H, _H), jnp.float32),
        compiler_params=pltpu.CompilerParams(
            dimension_semantics=("arbitrary",),
        ),
    )(x)

    gsize = int(0.1 * _H) // 2 * 2 + 1
    mask = _gauss_blur(im_diff.reshape(-1, 1, h, w), gsize, gsize / 3.0)
    mask = _ni_batch(mask.reshape(-1, h, w))
    activation = mask.reshape(b, -1) @ grid_cells.reshape(16, -1).T
    _, fg_index = jax.lax.top_k(activation, 8)
    sel = jax.nn.one_hot(fg_index, 16, dtype=jnp.int32).sum(axis=1)
    perm = jax.random.permutation(jax.random.key(42), b).astype(jnp.int32)
    srcb = jnp.where(sel > 0, jnp.arange(b, dtype=jnp.int32)[:, None],
                     perm[:, None])

    x6 = x.reshape(b, c * t, 4, 28, 4, 28)
    mesh = plsc.VectorSubcoreMesh(core_axis_name="c", subcore_axis_name="s")
    route = functools.partial(
        pl.kernel, mesh=mesh,
        out_type=jax.ShapeDtypeStruct((_B, _CT, 4, 28, 4, 28), jnp.float32),
        scratch_types=[
            pltpu.VMEM((16,), jnp.int32),
            pltpu.VMEM((_CT, 28, 28), jnp.float32),
            pltpu.VMEM((_CT, 28, 28), jnp.float32),
            pltpu.SemaphoreType.DMA,
            pltpu.SemaphoreType.DMA,
            pltpu.SemaphoreType.DMA,
            pltpu.SemaphoreType.DMA,
        ],
        compiler_params=pltpu.CompilerParams(use_tc_tiling_on_sc=False),
    )(_sc_route)
    out = route(x6, srcb)
    return out.reshape(b, c, t, h, w)
